# baseline (speedup 1.0000x reference)
import functools
import math

import jax
import jax.numpy as jnp
from jax.experimental import pallas as pl
from jax.experimental.pallas import tpu as pltpu

K = 8
LANES = 128
BLOCK_PAIRS = 1024
LN2 = math.log(2.0)


def _body(coef_ref, logit2_ref, x_ref, o_ref):
    x0 = x_ref[0::2, :]
    x1 = x_ref[1::2, :]

    l2 = logit2_ref[...]
    lm = jnp.max(l2, axis=1, keepdims=True)
    lse2 = lm + jnp.log2(jnp.sum(jnp.exp2(l2 - lm), axis=1, keepdims=True))

    x0sq = x0 * x0
    x1sq = x1 * x1
    lps = []
    for k in range(K):
        a0 = coef_ref[0, k]
        b0 = coef_ref[1, k]
        a1 = coef_ref[2, k]
        b1 = coef_ref[3, k]
        e = coef_ref[4, k]
        lps.append(a0 * x0sq + b0 * x0 + a1 * x1sq + b1 * x1 + e)
    m = lps[0]
    for k in range(1, K):
        m = jnp.maximum(m, lps[k])
    s = jnp.exp2(lps[0] - m)
    for k in range(1, K):
        s = s + jnp.exp2(lps[k] - m)
    o_ref[...] = (m + jnp.log2(s) - lse2) * LN2


@functools.partial(jax.jit, static_argnames=("pairs",))
def _run(coef, logit2, x2, pairs):
    grid = pairs // BLOCK_PAIRS
    return pl.pallas_call(
        _body,
        grid=(grid,),
        in_specs=[
            pl.BlockSpec(memory_space=pltpu.SMEM),
            pl.BlockSpec((1, K), lambda i: (0, 0)),
            pl.BlockSpec((2 * BLOCK_PAIRS, LANES), lambda i: (i, 0)),
        ],
        out_specs=pl.BlockSpec((BLOCK_PAIRS, LANES), lambda i: (i, 0)),
        out_shape=jax.ShapeDtypeStruct((pairs, LANES), jnp.float32),
    )(coef, logit2, x2)


def kernel(x, logits, means, scales):
    n = x.shape[0]
    log2e = 1.0 / LN2
    inv2 = 1.0 / (scales * scales)
    a = (-0.5 * log2e) * inv2
    b = log2e * (means * inv2)
    e = log2e * (logits - jnp.sum(jnp.log(scales), axis=1)
                 - math.log(2.0 * math.pi)
                 - 0.5 * jnp.sum(means * means * inv2, axis=1))
    coef = jnp.stack([a[:, 0], b[:, 0], a[:, 1], b[:, 1], e])
    logit2 = (log2e * logits).reshape(1, K)

    pairs = n // LANES
    x2 = x.reshape(pairs, LANES, 2).swapaxes(1, 2).reshape(2 * pairs, LANES)
    x2 = pltpu.with_memory_space_constraint(x2, pltpu.HBM)
    out = _run(coef, logit2, x2, pairs)
    return out.reshape(n)


# speedup vs baseline: 1.4581x; 1.4581x over previous
import functools
import math

import jax
import jax.numpy as jnp
from jax.experimental import pallas as pl
from jax.experimental.pallas import tpu as pltpu

K = 8
LANES = 128
BLOCK_PAIRS = 1024


def _body(coef_ref, x_ref, o_ref):
    x0 = x_ref[0::2, :]
    x1 = x_ref[1::2, :]

    x0sq = x0 * x0
    x1sq = x1 * x1
    lps = []
    for k in range(K):
        a0 = coef_ref[0, k]
        b0 = coef_ref[1, k]
        a1 = coef_ref[2, k]
        b1 = coef_ref[3, k]
        e = coef_ref[4, k]
        lps.append(a0 * x0sq + b0 * x0 + a1 * x1sq + b1 * x1 + e)
    m = lps[0]
    for k in range(1, K):
        m = jnp.maximum(m, lps[k])
    s = jnp.exp2(lps[0] - m)
    for k in range(1, K):
        s = s + jnp.exp2(lps[k] - m)
    o_ref[...] = (m + jnp.log2(s)) * math.log(2.0)


@functools.partial(jax.jit, static_argnames=("pairs",))
def _run(coef, x2, pairs):
    grid = pairs // BLOCK_PAIRS
    return pl.pallas_call(
        _body,
        grid=(grid,),
        in_specs=[
            pl.BlockSpec(memory_space=pltpu.SMEM),
            pl.BlockSpec((2 * BLOCK_PAIRS, LANES), lambda i: (i, 0)),
        ],
        out_specs=pl.BlockSpec((BLOCK_PAIRS, LANES), lambda i: (i, 0)),
        out_shape=jax.ShapeDtypeStruct((pairs, LANES), jnp.float32),
    )(coef, x2)


def kernel(x, logits, means, scales):
    n = x.shape[0]
    log2e = 1.0 / math.log(2.0)
    logw = jax.nn.log_softmax(logits)
    inv2 = 1.0 / (scales * scales)
    a = (-0.5 * log2e) * inv2
    b = log2e * (means * inv2)
    e = log2e * (logw - jnp.sum(jnp.log(scales), axis=1)
                 - math.log(2.0 * math.pi)
                 - 0.5 * jnp.sum(means * means * inv2, axis=1))
    coef = jnp.stack([a[:, 0], b[:, 0], a[:, 1], b[:, 1], e])

    pairs = n // LANES
    x2 = x.reshape(pairs, LANES, 2).swapaxes(1, 2).reshape(2 * pairs, LANES)
    x2 = pltpu.with_memory_space_constraint(x2, pltpu.HBM)
    out = _run(coef, x2, pairs)
    return out.reshape(n)
